# baseline (speedup 1.0000x reference)
import functools

import jax
import jax.numpy as jnp
from jax import lax
from jax.experimental import pallas as pl
from jax.experimental.pallas import tpu as pltpu
from jax.experimental.pallas import tpu_sc as plsc

_DIM = 64
_NC = 2
_NS = 16
_NW = _NC * _NS

_ROWS = 2
_LANES = 16


def _pe_concat_kernel(seq, rows_per_worker, x_ref, t_ref, pe2_ref, out_ref,
                      idx_v, outv, xv, sem):
    wid = lax.axis_index("s") * _NC + lax.axis_index("c")
    row_start = wid * rows_per_worker
    chunk = _ROWS * seq
    n_iters = rows_per_worker // _ROWS

    splits = [(o, min(128, chunk - o)) for o in range(0, chunk, 128)]

    def body(it, _):
        row = row_start + it * _ROWS
        base = row * seq
        pltpu.sync_copy(t_ref.at[pl.ds(base, chunk)], idx_v)
        copies = []
        for off, cnt in splits:
            copies.append(pltpu.async_copy(
                pe2_ref.at[idx_v.at[pl.ds(off, cnt)]],
                outv.at[pl.ds(off, cnt)],
                sem))
        copies.append(pltpu.async_copy(
            x_ref.at[pl.ds(base * _DIM, chunk * _DIM)], xv, sem))
        for c in copies:
            c.wait()

        def vbody(s, _):
            for j in range(_ROWS):
                r = j * seq + s
                for c in range(_DIM // _LANES):
                    outv[r, pl.ds(c * _LANES, _LANES)] = (
                        xv[pl.ds(r * _DIM + c * _LANES, _LANES)])
            return ()

        lax.fori_loop(0, seq, vbody, ())

        pltpu.sync_copy(outv, out_ref.at[pl.ds(base, chunk)])
        return ()

    lax.fori_loop(0, n_iters, body, ())


def kernel(x, t, pe):
    batch, seq, dim = x.shape
    n = batch * seq
    assert batch % (_NW * _ROWS) == 0
    rows_per_worker = batch // _NW

    x1 = x.reshape(n * dim)
    t1 = t.reshape(n)
    pe2 = jnp.concatenate([jnp.zeros_like(pe), pe], axis=1)

    mesh = plsc.VectorSubcoreMesh(core_axis_name="c", subcore_axis_name="s")
    out = pl.kernel(
        functools.partial(_pe_concat_kernel, seq, rows_per_worker),
        out_type=jax.ShapeDtypeStruct((n, 2 * dim), jnp.float32),
        mesh=mesh,
        scratch_types=[
            pltpu.VMEM((_ROWS * seq,), jnp.int32),
            pltpu.VMEM((_ROWS * seq, 2 * dim), jnp.float32),
            pltpu.VMEM((_ROWS * seq * dim,), jnp.float32),
            pltpu.SemaphoreType.DMA,
        ],
    )(x1, t1, pe2)
    return out.reshape(batch, seq, 2 * dim)


# speedup vs baseline: 1.7519x; 1.7519x over previous
import functools

import jax
import jax.numpy as jnp
from jax import lax
from jax.experimental import pallas as pl
from jax.experimental.pallas import tpu as pltpu
from jax.experimental.pallas import tpu_sc as plsc

_DIM = 64
_NC = 2
_NS = 16
_NW = _NC * _NS

_CHUNK = 400
_LANES = 16
_ROW_UNROLL = 8

_SPLITS = [(o, min(128, _CHUNK - o)) for o in range(0, _CHUNK, 128)]


def _pe_concat_kernel(n_iters, x_ref, t_ref, pe2_ref, out_ref,
                      idx_v, outv, xv, sem):
    wid = lax.axis_index("s") * _NC + lax.axis_index("c")
    start = wid * n_iters * _CHUNK

    def body(it, _):
        base = start + it * _CHUNK
        pltpu.sync_copy(t_ref.at[pl.ds(base, _CHUNK)], idx_v)
        copies = []
        for off, cnt in _SPLITS:
            copies.append(pltpu.async_copy(
                pe2_ref.at[idx_v.at[pl.ds(off, cnt)]],
                outv.at[pl.ds(off, cnt)],
                sem))
        copies.append(pltpu.async_copy(
            x_ref.at[pl.ds(base, _CHUNK)], xv, sem))
        for c in copies:
            c.wait()

        def vbody(k, _):
            r0 = k * _ROW_UNROLL
            for u in range(_ROW_UNROLL):
                for c in range(_DIM // _LANES):
                    outv[r0 + u, pl.ds(c * _LANES, _LANES)] = (
                        xv[r0 + u, pl.ds(c * _LANES, _LANES)])
            return ()

        lax.fori_loop(0, _CHUNK // _ROW_UNROLL, vbody, ())

        pltpu.sync_copy(outv, out_ref.at[pl.ds(base, _CHUNK)])
        return ()

    lax.fori_loop(0, n_iters, body, ())


def kernel(x, t, pe):
    batch, seq, dim = x.shape
    n = batch * seq
    assert n % (_NW * _CHUNK) == 0
    n_iters = n // (_NW * _CHUNK)

    x2 = x.reshape(n, dim)
    t1 = t.reshape(n)
    pe2 = jnp.concatenate([jnp.zeros_like(pe), pe], axis=1)

    mesh = plsc.VectorSubcoreMesh(core_axis_name="c", subcore_axis_name="s")
    out = pl.kernel(
        functools.partial(_pe_concat_kernel, n_iters),
        out_type=jax.ShapeDtypeStruct((n, 2 * dim), jnp.float32),
        mesh=mesh,
        scratch_types=[
            pltpu.VMEM((_CHUNK,), jnp.int32),
            pltpu.VMEM((_CHUNK, 2 * dim), jnp.float32),
            pltpu.VMEM((_CHUNK, dim), jnp.float32),
            pltpu.SemaphoreType.DMA,
        ],
    )(x2, t1, pe2)
    return out.reshape(batch, seq, 2 * dim)


# speedup vs baseline: 1.8960x; 1.0823x over previous
import functools

import jax
import jax.numpy as jnp
from jax import lax
from jax.experimental import pallas as pl
from jax.experimental.pallas import tpu as pltpu
from jax.experimental.pallas import tpu_sc as plsc

_DIM = 64
_NC = 2
_NS = 16
_NW = _NC * _NS

_CHUNK = 200
_LANES = 16
_ROW_UNROLL = 8

_SPLITS = [(o, min(128, _CHUNK - o)) for o in range(0, _CHUNK, 128)]


def _pe_concat_kernel(n_iters, x_ref, t_ref, pe2_ref, out_ref,
                      idx_a, outv_a, xv_a, idx_b, outv_b, xv_b,
                      sem_a, sem_b, sem_wa, sem_wb):
    wid = lax.axis_index("s") * _NC + lax.axis_index("c")
    start = wid * n_iters * 2 * _CHUNK

    def stage(base, idx_v, outv, xv, sem):
        pltpu.sync_copy(t_ref.at[pl.ds(base, _CHUNK)], idx_v)
        copies = []
        for off, cnt in _SPLITS:
            copies.append(pltpu.async_copy(
                pe2_ref.at[idx_v.at[pl.ds(off, cnt)]],
                outv.at[pl.ds(off, cnt)],
                sem))
        copies.append(pltpu.async_copy(
            x_ref.at[pl.ds(base, _CHUNK)], xv, sem))
        return copies

    def interleave(outv, xv):
        def vbody(k, _):
            r0 = k * _ROW_UNROLL
            for u in range(_ROW_UNROLL):
                for c in range(_DIM // _LANES):
                    outv[r0 + u, pl.ds(c * _LANES, _LANES)] = (
                        xv[r0 + u, pl.ds(c * _LANES, _LANES)])
            return ()
        lax.fori_loop(0, _CHUNK // _ROW_UNROLL, vbody, ())

    def body(it, _):
        base_a = start + it * 2 * _CHUNK
        base_b = base_a + _CHUNK

        @pl.when(it >= 1)
        def _drain_a():
            pltpu.make_async_copy(
                outv_a, out_ref.at[pl.ds(base_a - 2 * _CHUNK, _CHUNK)],
                sem_wa).wait()

        copies_a = stage(base_a, idx_a, outv_a, xv_a, sem_a)

        @pl.when(it >= 1)
        def _drain_b():
            pltpu.make_async_copy(
                outv_b, out_ref.at[pl.ds(base_b - 2 * _CHUNK, _CHUNK)],
                sem_wb).wait()

        copies_b = stage(base_b, idx_b, outv_b, xv_b, sem_b)

        for c in copies_a:
            c.wait()
        interleave(outv_a, xv_a)
        pltpu.async_copy(outv_a, out_ref.at[pl.ds(base_a, _CHUNK)], sem_wa)

        for c in copies_b:
            c.wait()
        interleave(outv_b, xv_b)
        pltpu.async_copy(outv_b, out_ref.at[pl.ds(base_b, _CHUNK)], sem_wb)
        return ()

    lax.fori_loop(0, n_iters, body, ())

    last_a = start + (n_iters - 1) * 2 * _CHUNK
    pltpu.make_async_copy(outv_a, out_ref.at[pl.ds(last_a, _CHUNK)],
                          sem_wa).wait()
    pltpu.make_async_copy(outv_b, out_ref.at[pl.ds(last_a + _CHUNK, _CHUNK)],
                          sem_wb).wait()


def kernel(x, t, pe):
    batch, seq, dim = x.shape
    n = batch * seq
    assert n % (_NW * 2 * _CHUNK) == 0
    n_iters = n // (_NW * 2 * _CHUNK)

    x2 = x.reshape(n, dim)
    t1 = t.reshape(n)
    pe2 = jnp.concatenate([jnp.zeros_like(pe), pe], axis=1)

    mesh = plsc.VectorSubcoreMesh(core_axis_name="c", subcore_axis_name="s")
    out = pl.kernel(
        functools.partial(_pe_concat_kernel, n_iters),
        out_type=jax.ShapeDtypeStruct((n, 2 * dim), jnp.float32),
        mesh=mesh,
        scratch_types=[
            pltpu.VMEM((_CHUNK,), jnp.int32),
            pltpu.VMEM((_CHUNK, 2 * dim), jnp.float32),
            pltpu.VMEM((_CHUNK, dim), jnp.float32),
            pltpu.VMEM((_CHUNK,), jnp.int32),
            pltpu.VMEM((_CHUNK, 2 * dim), jnp.float32),
            pltpu.VMEM((_CHUNK, dim), jnp.float32),
            pltpu.SemaphoreType.DMA,
            pltpu.SemaphoreType.DMA,
            pltpu.SemaphoreType.DMA,
            pltpu.SemaphoreType.DMA,
        ],
    )(x2, t1, pe2)
    return out.reshape(batch, seq, 2 * dim)
